# baseline (speedup 1.0000x reference)
import functools

import jax
import jax.numpy as jnp
from jax import lax
from jax.experimental import pallas as pl
from jax.experimental.pallas import tpu as pltpu
from jax.experimental.pallas import tpu_sc as plsc

BATCH = 16384
EMBED = 64
TILE_ROWS = 8
NUM_CORES = 2
NUM_SUBCORES = 16
NUM_WORKERS = NUM_CORES * NUM_SUBCORES
B_PER_W = BATCH // NUM_WORKERS
CHUNK = 64
N_CHUNKS = B_PER_W // CHUNK

_mesh = plsc.VectorSubcoreMesh(core_axis_name="c", subcore_axis_name="s")


def _lane_scalar(vec, lane):
    splat = jnp.take_along_axis(
        vec, jnp.broadcast_to(lane, (16,)), axis=0, mode="promise_in_bounds")
    return jnp.max(splat)


@functools.partial(
    pl.kernel,
    mesh=_mesh,
    out_type=jax.ShapeDtypeStruct((BATCH // 2, 2 * EMBED), jnp.float32),
    scratch_types=[
        pltpu.VMEM((B_PER_W,), jnp.int32),
        pltpu.VMEM((CHUNK * TILE_ROWS, EMBED), jnp.float32),
        pltpu.VMEM((B_PER_W // 2, 2 * EMBED), jnp.float32),
        pltpu.SemaphoreType.DMA,
    ],
    compiler_params=pltpu.CompilerParams(needs_layout_passes=False),
)
def _embed_lookup(idx_hbm, table_hbm, out2_hbm, idx_v, blocks_v, dst_v, sem):
    wid = lax.axis_index("s") * NUM_CORES + lax.axis_index("c")
    base = pl.multiple_of(wid * B_PER_W, B_PER_W)
    pltpu.sync_copy(idx_hbm.at[pl.ds(base, B_PER_W)], idx_v)

    for chunk in range(N_CHUNKS):
        c_off = chunk * CHUNK
        copies = []
        for n in range(CHUNK):
            if n % 16 == 0:
                rvec = idx_v[pl.ds(c_off + n, 16)]
            row8 = pl.multiple_of((_lane_scalar(rvec, n % 16) >> 3) * TILE_ROWS,
                                  TILE_ROWS)
            copies.append(pltpu.async_copy(
                table_hbm.at[pl.ds(row8, TILE_ROWS)],
                blocks_v.at[pl.ds(n * TILE_ROWS, TILE_ROWS)], sem))
        for c in copies:
            c.wait()

        def extract(n, carry):
            g = c_off + n
            rvec = idx_v[pl.ds((g >> 4) << 4, 16)]
            sidx = _lane_scalar(rvec, g & 15)
            sub = sidx & 7
            dst_row = g >> 1
            dst_col = (g & 1) * EMBED
            for c4 in range(EMBED // 16):
                dst_v[dst_row, pl.ds(dst_col + c4 * 16, 16)] = (
                    blocks_v[n * TILE_ROWS + sub, pl.ds(c4 * 16, 16)])
            return carry

        lax.fori_loop(0, CHUNK, extract, 0)

    pltpu.sync_copy(
        dst_v,
        out2_hbm.at[pl.ds(pl.multiple_of(base // 2, B_PER_W // 2),
                          B_PER_W // 2)])


def kernel(node_ids, table):
    out2 = _embed_lookup(node_ids.astype(jnp.int32), table)
    return out2.reshape(BATCH, EMBED)


# speedup vs baseline: 1.3118x; 1.3118x over previous
import functools

import jax
import jax.numpy as jnp
from jax import lax
from jax.experimental import pallas as pl
from jax.experimental.pallas import tpu as pltpu
from jax.experimental.pallas import tpu_sc as plsc

BATCH = 16384
EMBED = 64
LANE = 128
NUM_CORES = 2
NUM_SUBCORES = 16
NUM_WORKERS = NUM_CORES * NUM_SUBCORES
B_PER_W = BATCH // NUM_WORKERS
CHUNK = 4
N_CHUNKS = B_PER_W // CHUNK

_mesh = plsc.VectorSubcoreMesh(core_axis_name="c", subcore_axis_name="s")


def _lane_scalar(vec, lane):
    splat = jnp.take_along_axis(
        vec, jnp.broadcast_to(lane, (16,)), axis=0, mode="promise_in_bounds")
    return jnp.max(splat)


@functools.partial(
    pl.kernel,
    mesh=_mesh,
    out_type=jax.ShapeDtypeStruct((BATCH // 2, 2 * EMBED), jnp.float32),
    scratch_types=[
        pltpu.VMEM((B_PER_W,), jnp.int32),
        pltpu.VMEM((CHUNK * EMBED, LANE), jnp.float32),
        pltpu.VMEM((B_PER_W // 2, 2 * EMBED), jnp.float32),
        pltpu.SemaphoreType.DMA,
    ],
    compiler_params=pltpu.CompilerParams(needs_layout_passes=False),
)
def _embed_lookup(idx_hbm, tab_t_hbm, out2_hbm, idx_v, slab_v, dst_v, sem):
    wid = lax.axis_index("s") * NUM_CORES + lax.axis_index("c")
    base = pl.multiple_of(wid * B_PER_W, B_PER_W)
    pltpu.sync_copy(idx_hbm.at[pl.ds(base, B_PER_W)], idx_v)

    lanes = lax.iota(jnp.int32, 16)

    def body(chunk, carry):
        c_off = chunk * CHUNK
        scalars = []
        copies = []
        for n in range(CHUNK):
            g = c_off + n
            rvec = idx_v[pl.ds((g >> 4) << 4, 16)]
            sidx = _lane_scalar(rvec, g & 15)
            scalars.append(sidx)
            col0 = pl.multiple_of((sidx >> 7) * LANE, LANE)
            copies.append(pltpu.async_copy(
                tab_t_hbm.at[:, pl.ds(col0, LANE)],
                slab_v.at[pl.ds(n * EMBED, EMBED)], sem))
        for c in copies:
            c.wait()
        for n in range(CHUNK):
            g = c_off + n
            col = jnp.broadcast_to(scalars[n] & (LANE - 1), (16,))
            dst_row = g >> 1
            dst_col = (g & 1) * EMBED
            for k in range(EMBED // 16):
                vals = plsc.load_gather(
                    slab_v, [n * EMBED + k * 16 + lanes, col])
                dst_v[dst_row, pl.ds(dst_col + k * 16, 16)] = vals
        return carry

    lax.fori_loop(0, N_CHUNKS, body, 0)

    pltpu.sync_copy(
        dst_v,
        out2_hbm.at[pl.ds(pl.multiple_of(base // 2, B_PER_W // 2),
                          B_PER_W // 2)])


def kernel(node_ids, table):
    out2 = _embed_lookup(node_ids.astype(jnp.int32), table.T)
    return out2.reshape(BATCH, EMBED)


# speedup vs baseline: 1.5965x; 1.2171x over previous
import functools

import jax
import jax.numpy as jnp
from jax import lax
from jax.experimental import pallas as pl
from jax.experimental.pallas import tpu as pltpu
from jax.experimental.pallas import tpu_sc as plsc

BATCH = 16384
EMBED = 64
LANE = 128
NUM_CORES = 2
NUM_SUBCORES = 16
NUM_WORKERS = NUM_CORES * NUM_SUBCORES
B_PER_W = BATCH // NUM_WORKERS
CHUNK = 4
N_CHUNKS = B_PER_W // CHUNK
HALF_CHUNKS = N_CHUNKS // 2
HALF_ROWS = HALF_CHUNKS * CHUNK // 2

_mesh = plsc.VectorSubcoreMesh(core_axis_name="c", subcore_axis_name="s")


def _lane_scalar(vec, lane):
    splat = jnp.take_along_axis(
        vec, jnp.broadcast_to(lane, (16,)), axis=0, mode="promise_in_bounds")
    return jnp.max(splat)


@functools.partial(
    pl.kernel,
    mesh=_mesh,
    out_type=jax.ShapeDtypeStruct((BATCH // 2, 2 * EMBED), jnp.float32),
    scratch_types=[
        pltpu.VMEM((B_PER_W,), jnp.int32),
        pltpu.VMEM((CHUNK * EMBED, LANE), jnp.float32),
        pltpu.VMEM((CHUNK * EMBED, LANE), jnp.float32),
        pltpu.VMEM((HALF_ROWS, 2 * EMBED), jnp.float32),
        pltpu.SemaphoreType.DMA,
        pltpu.SemaphoreType.DMA,
    ],
    compiler_params=pltpu.CompilerParams(needs_layout_passes=False),
)
def _embed_lookup(idx_hbm, tab_t_hbm, out2_hbm, idx_v, slab_a, slab_b, dst_v,
                  sem_a, sem_b):
    wid = lax.axis_index("s") * NUM_CORES + lax.axis_index("c")
    base = pl.multiple_of(wid * B_PER_W, B_PER_W)
    pltpu.sync_copy(idx_hbm.at[pl.ds(base, B_PER_W)], idx_v)

    lanes = lax.iota(jnp.int32, 16)

    def scalar_idx(g):
        rvec = idx_v[pl.ds((g >> 4) << 4, 16)]
        return _lane_scalar(rvec, g & 15)

    def issue(k, buf, sem):
        for n in range(CHUNK):
            sidx = scalar_idx(k * CHUNK + n)
            col0 = pl.multiple_of((sidx >> 7) * LANE, LANE)
            pltpu.async_copy(tab_t_hbm.at[:, pl.ds(col0, LANE)],
                             buf.at[pl.ds(n * EMBED, EMBED)], sem)

    def drain(buf, sem):
        for n in range(CHUNK):
            pltpu.make_async_copy(tab_t_hbm.at[:, pl.ds(0, LANE)],
                                  buf.at[pl.ds(n * EMBED, EMBED)], sem).wait()

    def extract(k, buf, row_off):
        for n in range(CHUNK):
            g = k * CHUNK + n
            col = jnp.broadcast_to(scalar_idx(g) & (LANE - 1), (16,))
            dst_row = (g >> 1) - row_off
            dst_col = (g & 1) * EMBED
            for c4 in range(EMBED // 16):
                vals = plsc.load_gather(
                    buf, [n * EMBED + c4 * 16 + lanes, col])
                dst_v[dst_row, pl.ds(dst_col + c4 * 16, 16)] = vals

    for h in range(2):
        h_off = h * HALF_CHUNKS
        row_off = h * HALF_ROWS
        issue(h_off, slab_a, sem_a)

        def pair(p, carry):
            k0 = h_off + 2 * p
            issue(k0 + 1, slab_b, sem_b)
            drain(slab_a, sem_a)
            extract(k0, slab_a, row_off)

            @pl.when(p < HALF_CHUNKS // 2 - 1)
            def _():
                issue(k0 + 2, slab_a, sem_a)

            drain(slab_b, sem_b)
            extract(k0 + 1, slab_b, row_off)
            return carry

        lax.fori_loop(0, HALF_CHUNKS // 2, pair, 0)
        pltpu.sync_copy(
            dst_v,
            out2_hbm.at[pl.ds(
                pl.multiple_of(base // 2 + h * HALF_ROWS, HALF_ROWS),
                HALF_ROWS)])


def kernel(node_ids, table):
    out2 = _embed_lookup(node_ids.astype(jnp.int32), table.T)
    return out2.reshape(BATCH, EMBED)


# speedup vs baseline: 1.7374x; 1.0882x over previous
import functools

import jax
import jax.numpy as jnp
from jax import lax
from jax.experimental import pallas as pl
from jax.experimental.pallas import tpu as pltpu
from jax.experimental.pallas import tpu_sc as plsc

BATCH = 16384
EMBED = 64
LANE = 128
NUM_CORES = 2
NUM_SUBCORES = 16
NUM_WORKERS = NUM_CORES * NUM_SUBCORES
B_PER_W = BATCH // NUM_WORKERS
CHUNK = 4
N_CHUNKS = B_PER_W // CHUNK
HALF_CHUNKS = N_CHUNKS // 2
HALF_ROWS = HALF_CHUNKS * CHUNK // 2

_mesh = plsc.VectorSubcoreMesh(core_axis_name="c", subcore_axis_name="s")


def _lane_scalar(vec, lane):
    splat = jnp.take_along_axis(
        vec, jnp.broadcast_to(lane, (16,)), axis=0, mode="promise_in_bounds")
    return jnp.max(splat)


@functools.partial(
    pl.kernel,
    mesh=_mesh,
    out_type=jax.ShapeDtypeStruct((BATCH // 2, 2 * EMBED), jnp.float32),
    scratch_types=[
        pltpu.VMEM((B_PER_W,), jnp.int32),
        pltpu.VMEM((CHUNK * EMBED, LANE), jnp.float32),
        pltpu.VMEM((CHUNK * EMBED, LANE), jnp.float32),
        pltpu.VMEM((HALF_ROWS, 2 * EMBED), jnp.float32),
        [pltpu.SemaphoreType.DMA] * CHUNK,
        [pltpu.SemaphoreType.DMA] * CHUNK,
    ],
    compiler_params=pltpu.CompilerParams(needs_layout_passes=False),
)
def _embed_lookup(idx_hbm, tab_t_hbm, out2_hbm, idx_v, slab_a, slab_b, dst_v,
                  sem_a, sem_b):
    wid = lax.axis_index("s") * NUM_CORES + lax.axis_index("c")
    base = pl.multiple_of(wid * B_PER_W, B_PER_W)
    pltpu.sync_copy(idx_hbm.at[pl.ds(base, B_PER_W)], idx_v)

    lanes = lax.iota(jnp.int32, 16)

    def scalar_idx(g):
        rvec = idx_v[pl.ds((g >> 4) << 4, 16)]
        return _lane_scalar(rvec, g & 15)

    def issue(k, buf, sems):
        for n in range(CHUNK):
            sidx = scalar_idx(k * CHUNK + n)
            col0 = pl.multiple_of((sidx >> 7) * LANE, LANE)
            pltpu.async_copy(tab_t_hbm.at[:, pl.ds(col0, LANE)],
                             buf.at[pl.ds(n * EMBED, EMBED)], sems[n])

    def drain_extract(k, buf, sems, row_off):
        for n in range(CHUNK):
            pltpu.make_async_copy(tab_t_hbm.at[:, pl.ds(0, LANE)],
                                  buf.at[pl.ds(n * EMBED, EMBED)],
                                  sems[n]).wait()
            g = k * CHUNK + n
            col = jnp.broadcast_to(scalar_idx(g) & (LANE - 1), (16,))
            dst_row = (g >> 1) - row_off
            dst_col = (g & 1) * EMBED
            for c4 in range(EMBED // 16):
                vals = plsc.load_gather(
                    buf, [n * EMBED + c4 * 16 + lanes, col])
                dst_v[dst_row, pl.ds(dst_col + c4 * 16, 16)] = vals

    for h in range(2):
        h_off = h * HALF_CHUNKS
        row_off = h * HALF_ROWS
        issue(h_off, slab_a, sem_a)

        def pair(p, carry):
            k0 = h_off + 2 * p
            issue(k0 + 1, slab_b, sem_b)
            drain_extract(k0, slab_a, sem_a, row_off)

            @pl.when(p < HALF_CHUNKS // 2 - 1)
            def _():
                issue(k0 + 2, slab_a, sem_a)

            drain_extract(k0 + 1, slab_b, sem_b, row_off)
            return carry

        lax.fori_loop(0, HALF_CHUNKS // 2, pair, 0)
        pltpu.sync_copy(
            dst_v,
            out2_hbm.at[pl.ds(
                pl.multiple_of(base // 2 + h * HALF_ROWS, HALF_ROWS),
                HALF_ROWS)])


def kernel(node_ids, table):
    out2 = _embed_lookup(node_ids.astype(jnp.int32), table.T)
    return out2.reshape(BATCH, EMBED)


# speedup vs baseline: 1.9213x; 1.1058x over previous
import functools

import jax
import jax.numpy as jnp
from jax import lax
from jax.experimental import pallas as pl
from jax.experimental.pallas import tpu as pltpu
from jax.experimental.pallas import tpu_sc as plsc

BATCH = 16384
EMBED = 64
LANE = 128
NUM_CORES = 2
NUM_SUBCORES = 16
NUM_WORKERS = NUM_CORES * NUM_SUBCORES
B_PER_W = BATCH // NUM_WORKERS
RING = 8
N_GROUPS = B_PER_W // RING
HALF_GROUPS = N_GROUPS // 2
HALF_ROWS = B_PER_W // 4

_mesh = plsc.VectorSubcoreMesh(core_axis_name="c", subcore_axis_name="s")


def _lane_scalar(vec, lane):
    splat = jnp.take_along_axis(
        vec, jnp.broadcast_to(lane, (16,)), axis=0, mode="promise_in_bounds")
    return jnp.max(splat)


@functools.partial(
    pl.kernel,
    mesh=_mesh,
    out_type=jax.ShapeDtypeStruct((BATCH // 2, 2 * EMBED), jnp.float32),
    scratch_types=[
        pltpu.VMEM((B_PER_W,), jnp.int32),
        pltpu.VMEM((RING * EMBED, LANE), jnp.float32),
        pltpu.VMEM((HALF_ROWS, 2 * EMBED), jnp.float32),
        [pltpu.SemaphoreType.DMA] * RING,
    ],
    compiler_params=pltpu.CompilerParams(needs_layout_passes=False),
)
def _embed_lookup(idx_hbm, tab_t_hbm, out2_hbm, idx_v, slab_v, dst_v, sems):
    wid = lax.axis_index("s") * NUM_CORES + lax.axis_index("c")
    base = pl.multiple_of(wid * B_PER_W, B_PER_W)
    pltpu.sync_copy(idx_hbm.at[pl.ds(base, B_PER_W)], idx_v)

    lanes = lax.iota(jnp.int32, 16)

    def scalar_idx(g):
        rvec = idx_v[pl.ds((g >> 4) << 4, 16)]
        return _lane_scalar(rvec, g & 15)

    def issue(g, slot):
        col0 = pl.multiple_of((scalar_idx(g) >> 7) * LANE, LANE)
        pltpu.async_copy(tab_t_hbm.at[:, pl.ds(col0, LANE)],
                         slab_v.at[pl.ds(slot * EMBED, EMBED)], sems[slot])

    for n in range(RING):
        issue(n, n)

    for h in range(2):
        row_off = h * HALF_ROWS

        def group(p, carry):
            grp = h * HALF_GROUPS + p
            for n in range(RING):
                g = grp * RING + n
                pltpu.make_async_copy(
                    tab_t_hbm.at[:, pl.ds(0, LANE)],
                    slab_v.at[pl.ds(n * EMBED, EMBED)], sems[n]).wait()
                col = jnp.broadcast_to(scalar_idx(g) & (LANE - 1), (16,))
                dst_row = (g >> 1) - row_off
                dst_col = (g & 1) * EMBED
                for c4 in range(EMBED // 16):
                    vals = plsc.load_gather(
                        slab_v, [n * EMBED + c4 * 16 + lanes, col])
                    dst_v[dst_row, pl.ds(dst_col + c4 * 16, 16)] = vals

                @pl.when(g + RING < B_PER_W)
                def _():
                    issue(g + RING, n)
            return carry

        lax.fori_loop(0, HALF_GROUPS, group, 0)
        pltpu.sync_copy(
            dst_v,
            out2_hbm.at[pl.ds(
                pl.multiple_of(base // 2 + h * HALF_ROWS, HALF_ROWS),
                HALF_ROWS)])


def kernel(node_ids, table):
    out2 = _embed_lookup(node_ids.astype(jnp.int32), table.T)
    return out2.reshape(BATCH, EMBED)
